# Initial kernel scaffold; baseline (speedup 1.0000x reference)
import jax
import jax.numpy as jnp
from jax.experimental import pallas as pl


def kernel(x, codebook):
    raise NotImplementedError("write your pallas kernel here")


# speedup vs baseline: 1.2215x; 1.2215x over previous
import functools

import jax
import jax.numpy as jnp
from jax import lax
from jax.experimental import pallas as pl
from jax.experimental.pallas import tpu as pltpu
from jax.experimental.pallas import tpu_sc as plsc

_SC_NUM_CORES = 2
_SC_NUM_SUBCORES = 16
_SC_NUM_WORKERS = _SC_NUM_CORES * _SC_NUM_SUBCORES

_BLOCK_M = 512
_BLOCK_K = 2048

_GATHER_CHUNK = 128
_GATHER_GROUP = 8


def _argmin_body(x_ref, sx_ref, cb_ref, sc_ref, idx_ref, loss_ref):
    bm = x_ref.shape[0]
    n_codes = cb_ref.shape[0]
    n_chunks = n_codes // _BLOCK_K

    x = x_ref[...]
    sx = sx_ref[...]

    def chunk(k, carry):
        run_min, run_idx = carry
        cb = cb_ref[pl.ds(k * _BLOCK_K, _BLOCK_K), :]
        sc = sc_ref[:, pl.ds(k * _BLOCK_K, _BLOCK_K)]
        mm = lax.dot_general(x, cb, (((1,), (1,)), ((), ())),
                             preferred_element_type=jnp.float32)
        dist = (sx + sc) - 2.0 * mm
        cmin = jnp.min(dist, axis=1, keepdims=True)
        iot = lax.broadcasted_iota(jnp.int32, dist.shape, 1)
        cidx = jnp.min(jnp.where(dist == cmin, iot, n_codes),
                       axis=1, keepdims=True) + k * _BLOCK_K
        better = cmin < run_min
        return (jnp.where(better, cmin, run_min),
                jnp.where(better, cidx, run_idx))

    init = (jnp.full((bm, 1), jnp.inf, jnp.float32),
            jnp.zeros((bm, 1), jnp.int32))
    run_min, run_idx = lax.fori_loop(0, n_chunks, chunk, init)
    idx_ref[...] = run_idx
    loss_ref[...] = jnp.reshape(jnp.sum(run_min), (1, 1, 1))


def _tc_argmin(x, codebook, sx, sc):
    n, g = x.shape
    n_blocks = n // _BLOCK_M
    grid = (n_blocks,)
    idx, loss_parts = pl.pallas_call(
        _argmin_body,
        grid=grid,
        in_specs=[
            pl.BlockSpec((_BLOCK_M, g), lambda i: (i, 0)),
            pl.BlockSpec((_BLOCK_M, 1), lambda i: (i, 0)),
            pl.BlockSpec(codebook.shape, lambda i: (0, 0)),
            pl.BlockSpec(sc.shape, lambda i: (0, 0)),
        ],
        out_specs=[
            pl.BlockSpec((_BLOCK_M, 1), lambda i: (i, 0)),
            pl.BlockSpec((1, 1, 1), lambda i: (i, 0, 0)),
        ],
        out_shape=[
            jax.ShapeDtypeStruct((n, 1), jnp.int32),
            jax.ShapeDtypeStruct((n_blocks, 1, 1), jnp.float32),
        ],
    )(x, sx, codebook, sc)
    return idx, loss_parts


def _sc_gather(codebook, indices):
    b = indices.shape[0]
    d = codebook.shape[1]
    b_per_w = b // _SC_NUM_WORKERS
    chunks_per_w = b_per_w // _GATHER_CHUNK
    group_rows = _GATHER_CHUNK * _GATHER_GROUP
    n_groups = b_per_w // group_rows
    idx2 = indices.reshape(b // _GATHER_CHUNK, _GATHER_CHUNK)
    mesh = plsc.VectorSubcoreMesh(core_axis_name="c", subcore_axis_name="s")

    @functools.partial(
        pl.kernel,
        mesh=mesh,
        out_type=jax.ShapeDtypeStruct((b, d), jnp.float32),
        scratch_types=[
            pltpu.VMEM((chunks_per_w, _GATHER_CHUNK), jnp.int32),
            pltpu.VMEM((group_rows, d), jnp.float32),
            pltpu.SemaphoreType.DMA,
        ],
        compiler_params=pltpu.CompilerParams(use_tc_tiling_on_sc=False),
    )
    def gk(cb_hbm, idx_hbm, out_hbm, idx_v, rows_v, sem):
        wid = lax.axis_index("s") * _SC_NUM_CORES + lax.axis_index("c")
        base = wid * b_per_w
        pltpu.sync_copy(idx_hbm.at[pl.ds(wid * chunks_per_w, chunks_per_w)],
                        idx_v)

        def group(gi, _):
            for c in range(_GATHER_GROUP):
                pltpu.async_copy(
                    cb_hbm.at[idx_v.at[gi * _GATHER_GROUP + c]],
                    rows_v.at[pl.ds(c * _GATHER_CHUNK, _GATHER_CHUNK)],
                    sem)
            pltpu.make_async_copy(
                cb_hbm.at[pl.ds(0, group_rows)], rows_v, sem).wait()
            pltpu.sync_copy(
                rows_v, out_hbm.at[pl.ds(base + gi * group_rows, group_rows)])
            return _

        lax.fori_loop(0, n_groups, group, 0)

    return gk(codebook, idx2)


def kernel(x, codebook):
    sx = (x ** 2).sum(axis=-1, keepdims=True)
    sc = (codebook ** 2).sum(axis=-1).reshape(1, -1)
    idx2, loss_parts = _tc_argmin(x, codebook, sx, sc)
    indices = idx2.reshape(-1)
    quantized = _sc_gather(codebook, indices)
    commit_loss = loss_parts.sum() / jnp.float32(x.size)
    return (quantized, indices, commit_loss)


# speedup vs baseline: 1.2483x; 1.0219x over previous
import functools

import jax
import jax.numpy as jnp
from jax import lax
from jax.experimental import pallas as pl
from jax.experimental.pallas import tpu as pltpu
from jax.experimental.pallas import tpu_sc as plsc

_SC_NUM_CORES = 2
_SC_NUM_SUBCORES = 16
_SC_NUM_WORKERS = _SC_NUM_CORES * _SC_NUM_SUBCORES

_BLOCK_M = 512
_BLOCK_K = 2048

_GATHER_CHUNK = 128
_GATHER_GROUP = 8


def _argmin_body(x2_ref, sx_ref, cb_ref, sc_ref, idx_ref, loss_ref):
    bm = x2_ref.shape[0]
    n_codes = cb_ref.shape[0]
    n_chunks = n_codes // _BLOCK_K

    x2 = x2_ref[...] * jnp.float32(-2.0)
    sx = sx_ref[...]
    iotf = lax.broadcasted_iota(jnp.int32, (bm, _BLOCK_K), 1).astype(jnp.float32)

    def chunk(k, carry):
        run_min, run_idx = carry
        cb = cb_ref[pl.ds(k * _BLOCK_K, _BLOCK_K), :]
        sc = sc_ref[:, pl.ds(k * _BLOCK_K, _BLOCK_K)]
        mm2 = lax.dot_general(x2, cb, (((1,), (1,)), ((), ())),
                              preferred_element_type=jnp.float32)
        dist = (sx + sc) + mm2
        cmin = jnp.min(dist, axis=1, keepdims=True)
        cidx = jnp.min(jnp.where(dist == cmin, iotf, jnp.float32(n_codes)),
                       axis=1, keepdims=True) + k * jnp.float32(_BLOCK_K)
        better = cmin < run_min
        return (jnp.where(better, cmin, run_min),
                jnp.where(better, cidx, run_idx))

    init = (jnp.full((bm, 1), jnp.inf, jnp.float32),
            jnp.zeros((bm, 1), jnp.float32))
    run_min, run_idx = lax.fori_loop(0, n_chunks, chunk, init)
    idx_ref[...] = run_idx.astype(jnp.int32)
    loss_ref[...] = jnp.reshape(jnp.sum(run_min), (1, 1, 1))


def _tc_argmin(x, codebook, sx, sc):
    n, g = x.shape
    n_blocks = n // _BLOCK_M
    grid = (n_blocks,)
    idx, loss_parts = pl.pallas_call(
        _argmin_body,
        grid=grid,
        in_specs=[
            pl.BlockSpec((_BLOCK_M, g), lambda i: (i, 0)),
            pl.BlockSpec((_BLOCK_M, 1), lambda i: (i, 0)),
            pl.BlockSpec(codebook.shape, lambda i: (0, 0)),
            pl.BlockSpec(sc.shape, lambda i: (0, 0)),
        ],
        out_specs=[
            pl.BlockSpec((_BLOCK_M, 1), lambda i: (i, 0)),
            pl.BlockSpec((1, 1, 1), lambda i: (i, 0, 0)),
        ],
        out_shape=[
            jax.ShapeDtypeStruct((n, 1), jnp.int32),
            jax.ShapeDtypeStruct((n_blocks, 1, 1), jnp.float32),
        ],
    )(x, sx, codebook, sc)
    return idx, loss_parts


def _sc_gather(codebook, indices):
    b = indices.shape[0]
    d = codebook.shape[1]
    b_per_w = b // _SC_NUM_WORKERS
    chunks_per_w = b_per_w // _GATHER_CHUNK
    group_rows = _GATHER_CHUNK * _GATHER_GROUP
    n_groups = b_per_w // group_rows
    idx2 = indices.reshape(b // _GATHER_CHUNK, _GATHER_CHUNK)
    mesh = plsc.VectorSubcoreMesh(core_axis_name="c", subcore_axis_name="s")

    @functools.partial(
        pl.kernel,
        mesh=mesh,
        out_type=jax.ShapeDtypeStruct((b, d), jnp.float32),
        scratch_types=[
            pltpu.VMEM((chunks_per_w, _GATHER_CHUNK), jnp.int32),
            pltpu.VMEM((group_rows, d), jnp.float32),
            pltpu.SemaphoreType.DMA,
        ],
        compiler_params=pltpu.CompilerParams(use_tc_tiling_on_sc=False),
    )
    def gk(cb_hbm, idx_hbm, out_hbm, idx_v, rows_v, sem):
        wid = lax.axis_index("s") * _SC_NUM_CORES + lax.axis_index("c")
        base = wid * b_per_w
        pltpu.sync_copy(idx_hbm.at[pl.ds(wid * chunks_per_w, chunks_per_w)],
                        idx_v)

        def group(gi, _):
            for c in range(_GATHER_GROUP):
                pltpu.async_copy(
                    cb_hbm.at[idx_v.at[gi * _GATHER_GROUP + c]],
                    rows_v.at[pl.ds(c * _GATHER_CHUNK, _GATHER_CHUNK)],
                    sem)
            pltpu.make_async_copy(
                cb_hbm.at[pl.ds(0, group_rows)], rows_v, sem).wait()
            pltpu.sync_copy(
                rows_v, out_hbm.at[pl.ds(base + gi * group_rows, group_rows)])
            return _

        lax.fori_loop(0, n_groups, group, 0)

    return gk(codebook, idx2)


def kernel(x, codebook):
    sx = (x ** 2).sum(axis=-1, keepdims=True)
    sc = (codebook ** 2).sum(axis=-1).reshape(1, -1)
    idx2, loss_parts = _tc_argmin(x, codebook, sx, sc)
    indices = idx2.reshape(-1)
    quantized = _sc_gather(codebook, indices)
    commit_loss = loss_parts.sum() / jnp.float32(x.size)
    return (quantized, indices, commit_loss)
